# Initial kernel scaffold; baseline (speedup 1.0000x reference)
import jax
import jax.numpy as jnp
from jax.experimental import pallas as pl


def kernel(x, edge_index, batch, W_pre, b_pre, W1_0, b1_0, W2_0, b2_0, mem_0, W1_1, b1_1, W2_1, b2_1, mem_1, W_lin, b_lin):
    raise NotImplementedError("write your pallas kernel here")


# speedup vs baseline: 5.0252x; 5.0252x over previous
import functools

import jax
import jax.numpy as jnp
from jax import lax
from jax.experimental import pallas as pl
from jax.experimental.pallas import tpu as pltpu
from jax.experimental.pallas import tpu_sc as plsc

N = 10000
E = 320000
NFEAT = 128
NHID = 128
NCLASS = 10
NMEM = 8
NGRAPH = 64

NP = 10240
CPT = 80
EP = 32 * CPT * 128
BLK = 256
NBLK = NP // BLK

_NEG = -3.0e38


@functools.lru_cache(maxsize=None)
def _make_segsum(W):
    rows_per_tile = NP // 16

    mesh = plsc.VectorSubcoreMesh(core_axis_name="c", subcore_axis_name="s")

    @functools.partial(
        pl.kernel,
        out_type=jax.ShapeDtypeStruct((2, NP, W), jnp.float32),
        mesh=mesh,
        scratch_types=[
            pltpu.VMEM((CPT // 2, 128), jnp.int32),
            pltpu.VMEM((CPT // 2, 128), jnp.int32),
            pltpu.VMEM((2, 128, W), jnp.float32),
            pltpu.VMEM_SHARED((NP, W), jnp.float32),
            pltpu.SemaphoreType.DMA,
            pltpu.SemaphoreType.DMA,
        ],
        compiler_params=pltpu.CompilerParams(use_tc_tiling_on_sc=(W == 128)),
    )
    def segsum(h_hbm, src_hbm, dst_hbm, out_hbm, src_v, dst_v, rows_v,
               acc_sh, sem0, sem1):
        cid = lax.axis_index("c")
        sid = lax.axis_index("s")
        tid = cid * 16 + sid
        base = tid * CPT
        half = CPT // 2

        def _zrow(r, _):
            for j in range(W // 16):
                rows_v[0, r, pl.ds(j * 16, 16)] = jnp.zeros((16,), jnp.float32)
            return 0
        lax.fori_loop(0, 128, _zrow, 0)
        for kk in range(rows_per_tile // 128):
            pltpu.sync_copy(
                rows_v.at[0],
                acc_sh.at[pl.ds(sid * rows_per_tile + kk * 128, 128)])
        plsc.subcore_barrier()

        sems = (sem0, sem1)

        def _start(j, b):
            pltpu.async_copy(h_hbm.at[src_v.at[j]], rows_v.at[b], sems[b])

        def _wait(j, b):
            pltpu.make_async_copy(h_hbm.at[src_v.at[j]], rows_v.at[b],
                                  sems[b]).wait()

        def _scat(j, b):
            pltpu.sync_copy(rows_v.at[b], acc_sh.at[dst_v.at[j]], add=True)

        for hh in range(2):
            pltpu.sync_copy(src_hbm.at[pl.ds(base + hh * half, half)], src_v)
            pltpu.sync_copy(dst_hbm.at[pl.ds(base + hh * half, half)], dst_v)
            _start(0, 0)

            def _body(i, _):
                jj = 2 * i
                _start(jj + 1, 1)
                _wait(jj, 0)
                _scat(jj, 0)
                _start(jj + 2, 0)
                _wait(jj + 1, 1)
                _scat(jj + 1, 1)
                return 0
            lax.fori_loop(0, half // 2 - 1, _body, 0)
            _start(half - 1, 1)
            _wait(half - 2, 0)
            _scat(half - 2, 0)
            _wait(half - 1, 1)
            _scat(half - 1, 1)

        plsc.subcore_barrier()
        pltpu.sync_copy(
            acc_sh.at[pl.ds(sid * rows_per_tile, rows_per_tile)],
            out_hbm.at[cid, pl.ds(sid * rows_per_tile, rows_per_tile)])

    return segsum


def _pre_body(x_ref, w_ref, b_ref, o_ref):
    o_ref[...] = jnp.dot(x_ref[...], w_ref[...],
                         preferred_element_type=jnp.float32) + b_ref[...]


def _tc_pre(xp, W_pre, b_pre):
    return pl.pallas_call(
        _pre_body,
        grid=(NBLK,),
        in_specs=[
            pl.BlockSpec((BLK, NFEAT), lambda i: (i, 0)),
            pl.BlockSpec((NFEAT, NHID), lambda i: (0, 0)),
            pl.BlockSpec((1, NHID), lambda i: (0, 0)),
        ],
        out_specs=pl.BlockSpec((BLK, NHID), lambda i: (i, 0)),
        out_shape=jax.ShapeDtypeStruct((NP, NHID), jnp.float32),
    )(xp, W_pre, b_pre)


def _softmax8(t):
    m = jnp.max(t, axis=-1, keepdims=True)
    e = jnp.exp(t - m)
    return e / jnp.sum(e, axis=-1, keepdims=True)


def _layer0_body(h_ref, p0_ref, p1_ref, w1_ref, b1_ref, w2_ref, b2_ref,
                 mem_ref, o_ref):
    h = h_ref[...]
    agg = p0_ref[...] + p1_ref[...]
    z = jnp.concatenate([h, agg], axis=1)
    z = jnp.maximum(jnp.dot(z, w1_ref[...],
                            preferred_element_type=jnp.float32) + b1_ref[...],
                    0.0)
    z = jnp.dot(z, w2_ref[...], preferred_element_type=jnp.float32) + b2_ref[...]
    z = jnp.maximum(z, 0.0)
    t = lax.dot_general(z, mem_ref[...], (((1,), (1,)), ((), ())),
                        preferred_element_type=jnp.float32)
    s = _softmax8(t)
    o_ref[...] = jnp.concatenate([s, jnp.zeros_like(s)], axis=1)


def _tc_layer0(h, p0, p1, W1, b1, W2, b2, mem):
    return pl.pallas_call(
        _layer0_body,
        grid=(NBLK,),
        in_specs=[
            pl.BlockSpec((BLK, NHID), lambda i: (i, 0)),
            pl.BlockSpec((BLK, NHID), lambda i: (i, 0)),
            pl.BlockSpec((BLK, NHID), lambda i: (i, 0)),
            pl.BlockSpec((2 * NHID, NHID), lambda i: (0, 0)),
            pl.BlockSpec((1, NHID), lambda i: (0, 0)),
            pl.BlockSpec((NHID, NHID), lambda i: (0, 0)),
            pl.BlockSpec((1, NHID), lambda i: (0, 0)),
            pl.BlockSpec((NMEM, NHID), lambda i: (0, 0)),
        ],
        out_specs=pl.BlockSpec((BLK, 16), lambda i: (i, 0)),
        out_shape=jax.ShapeDtypeStruct((NP, 16), jnp.float32),
    )(h, p0, p1, W1, b1, W2, b2, mem)


def _final_body(s0_ref, q0_ref, q1_ref, bat_ref, mem0_ref, w1_ref, b1_ref,
                w2_ref, b2_ref, mem1_ref, wl_ref, bl_ref, o_ref,
                max_ref, sum_ref, cnt_ref):
    i = pl.program_id(0)

    @pl.when(i == 0)
    def _init():
        max_ref[...] = jnp.full_like(max_ref, _NEG)
        sum_ref[...] = jnp.zeros_like(sum_ref)
        cnt_ref[...] = jnp.zeros_like(cnt_ref)

    s0 = s0_ref[...][:, :NMEM]
    a8 = (q0_ref[...] + q1_ref[...])[:, :NMEM]
    mem0 = mem0_ref[...]
    h1 = jnp.dot(s0, mem0, preferred_element_type=jnp.float32)
    agg1 = jnp.dot(a8, mem0, preferred_element_type=jnp.float32)
    z = jnp.concatenate([h1, agg1], axis=1)
    z = jnp.maximum(jnp.dot(z, w1_ref[...],
                            preferred_element_type=jnp.float32) + b1_ref[...],
                    0.0)
    z = jnp.dot(z, w2_ref[...], preferred_element_type=jnp.float32) + b2_ref[...]
    z = jnp.maximum(z, 0.0)
    t = lax.dot_general(z, mem1_ref[...], (((1,), (1,)), ((), ())),
                        preferred_element_type=jnp.float32)
    s1 = _softmax8(t)
    h2 = jnp.dot(s1, mem1_ref[...], preferred_element_type=jnp.float32)
    hcat = jnp.concatenate([h1, h2], axis=1)

    bat = bat_ref[...]
    P = (lax.broadcasted_iota(jnp.int32, (BLK, NGRAPH), 1) == bat)
    Pf = P.astype(jnp.float32)
    sum_ref[...] += lax.dot_general(Pf, hcat, (((0,), (0,)), ((), ())),
                                    preferred_element_type=jnp.float32)
    cnt_ref[...] += lax.dot_general(
        Pf, jnp.ones((BLK, 128), jnp.float32), (((0,), (0,)), ((), ())),
        preferred_element_type=jnp.float32)

    gmin = jnp.min(bat)
    gmax = jnp.max(bat)
    for g in range(NGRAPH):
        @pl.when((g >= gmin) & (g <= gmax))
        def _upd():
            m = jnp.max(jnp.where(bat == g, hcat, _NEG), axis=0,
                        keepdims=True)
            max_ref[g:g + 1, :] = jnp.maximum(max_ref[g:g + 1, :], m)

    @pl.when(i == NBLK - 1)
    def _fin():
        out1 = max_ref[...]
        out1 = jnp.where(out1 > _NEG * 0.5, out1, 0.0)
        cnt = cnt_ref[...][:, 0:1]
        out2 = sum_ref[...] / jnp.maximum(cnt, 1.0)
        gfeat = jnp.concatenate([out1, out2], axis=1)
        logits = jnp.dot(gfeat, wl_ref[...],
                         preferred_element_type=jnp.float32) + bl_ref[...]
        m = jnp.max(logits, axis=-1, keepdims=True)
        lse = m + jnp.log(jnp.sum(jnp.exp(logits - m), axis=-1, keepdims=True))
        o_ref[...] = logits - lse


def _tc_final(s0p, q0, q1, batp, mem0, W1, b1, W2, b2, mem1, Wl, bl):
    return pl.pallas_call(
        _final_body,
        grid=(NBLK,),
        in_specs=[
            pl.BlockSpec((BLK, 16), lambda i: (i, 0)),
            pl.BlockSpec((BLK, 16), lambda i: (i, 0)),
            pl.BlockSpec((BLK, 16), lambda i: (i, 0)),
            pl.BlockSpec((BLK, 1), lambda i: (i, 0)),
            pl.BlockSpec((NMEM, NHID), lambda i: (0, 0)),
            pl.BlockSpec((2 * NHID, NHID), lambda i: (0, 0)),
            pl.BlockSpec((1, NHID), lambda i: (0, 0)),
            pl.BlockSpec((NHID, NHID), lambda i: (0, 0)),
            pl.BlockSpec((1, NHID), lambda i: (0, 0)),
            pl.BlockSpec((NMEM, NHID), lambda i: (0, 0)),
            pl.BlockSpec((2 * NHID * 2, NCLASS), lambda i: (0, 0)),
            pl.BlockSpec((1, NCLASS), lambda i: (0, 0)),
        ],
        out_specs=pl.BlockSpec((NGRAPH, NCLASS), lambda i: (0, 0)),
        out_shape=jax.ShapeDtypeStruct((NGRAPH, NCLASS), jnp.float32),
        scratch_shapes=[
            pltpu.VMEM((NGRAPH, 256), jnp.float32),
            pltpu.VMEM((NGRAPH, 256), jnp.float32),
            pltpu.VMEM((NGRAPH, 128), jnp.float32),
        ],
    )(s0p, q0, q1, batp, mem0, W1, b1, W2, b2, mem1, Wl, bl)


def kernel(x, edge_index, batch, W_pre, b_pre, W1_0, b1_0, W2_0, b2_0, mem_0,
           W1_1, b1_1, W2_1, b2_1, mem_1, W_lin, b_lin):
    src = edge_index[0]
    dst = edge_index[1]
    src2d = jnp.pad(src, (0, EP - E)).reshape(EP // 128, 128)
    dst2d = jnp.pad(dst, (0, EP - E), constant_values=N).reshape(EP // 128, 128)
    xp = jnp.pad(x, ((0, NP - N), (0, 0)))
    batp = jnp.pad(batch, (0, NP - N),
                   constant_values=NGRAPH).reshape(NP, 1).astype(jnp.int32)

    h = _tc_pre(xp, W_pre, b_pre.reshape(1, NHID))
    p = _make_segsum(NHID)(h, src2d, dst2d)
    s0p = _tc_layer0(h, p[0], p[1], W1_0, b1_0.reshape(1, NHID),
                     W2_0, b2_0.reshape(1, NHID), mem_0)
    q = _make_segsum(16)(s0p, src2d, dst2d)
    return _tc_final(s0p, q[0], q[1], batp, mem_0,
                     W1_1, b1_1.reshape(1, NHID), W2_1, b2_1.reshape(1, NHID),
                     mem_1, W_lin, b_lin.reshape(1, NCLASS))
